# baseline (speedup 1.0000x reference)
import functools

import jax
import jax.numpy as jnp
from jax import lax
from jax.experimental import pallas as pl
from jax.experimental.pallas import tpu as pltpu
from jax.experimental.pallas import tpu_sc as plsc

B = 16384
D = 64
NC = 2
NS = 16
NW = NC * NS
BPW = B // NW
L = 16
BROWS = 65536


def _mf_body(u_hbm, v_hbm, ue_hbm, ie_hbm, ub_hbm, ib_hbm, out_hbm,
             uidx, vidx, ubg, vbg, urows, vrows, bub, bvb, obuf, pacc,
             sem):
    wid = lax.axis_index("s") * NC + lax.axis_index("c")
    base = wid * BPW

    pltpu.sync_copy(u_hbm.at[pl.ds(base, BPW)], uidx)
    pltpu.sync_copy(v_hbm.at[pl.ds(base, BPW)], vidx)

    cu = pltpu.async_copy(ue_hbm.at[uidx], urows, sem)
    cv = pltpu.async_copy(ie_hbm.at[vidx], vrows, sem)

    def shift(k, carry):
        ubg[pl.ds(k * L, L)] = lax.shift_right_logical(uidx[pl.ds(k * L, L)], 4)
        vbg[pl.ds(k * L, L)] = lax.shift_right_logical(vidx[pl.ds(k * L, L)], 4)
        return carry

    lax.fori_loop(0, BPW // L, shift, 0)

    cbu = pltpu.async_copy(ub_hbm.at[ubg], bub, sem)
    cbv = pltpu.async_copy(ib_hbm.at[vbg], bvb, sem)
    cu.wait()
    cv.wait()
    cbu.wait()
    cbv.wait()

    lanes = lax.iota(jnp.int32, L)
    lmask = jnp.full((L,), 15, jnp.int32)

    def block(kb, carry):
        for r in range(L):
            rr = kb * L + r
            acc = urows[rr, pl.ds(0, L)] * vrows[rr, pl.ds(0, L)]
            for c in range(1, D // L):
                acc = acc + urows[rr, pl.ds(c * L, L)] * vrows[rr, pl.ds(c * L, L)]
            pacc[pl.ds(r * L, L)] = acc
        tot = plsc.load_gather(pacc, [lanes * L])
        for t in range(1, L):
            tot = tot + plsc.load_gather(pacc, [lanes * L + t])
        rows = kb * L + lanes
        ulan = jnp.bitwise_and(uidx[pl.ds(kb * L, L)], lmask)
        vlan = jnp.bitwise_and(vidx[pl.ds(kb * L, L)], lmask)
        x = tot + plsc.load_gather(bub, [rows, ulan]) \
            + plsc.load_gather(bvb, [rows, vlan])
        obuf[pl.ds(kb * L, L)] = 1.0 / (1.0 + jnp.exp(-x))
        return carry

    lax.fori_loop(0, BPW // L, block, 0)

    pltpu.sync_copy(obuf, out_hbm.at[pl.ds(base, BPW)])


@jax.jit
def _mf(u, v, user_emb, item_emb, user_bias, item_bias):
    mesh = plsc.VectorSubcoreMesh(core_axis_name="c", subcore_axis_name="s")
    run = functools.partial(
        pl.kernel,
        mesh=mesh,
        out_type=jax.ShapeDtypeStruct((B,), jnp.float32),
        scratch_types=[
            pltpu.VMEM((BPW,), jnp.int32),
            pltpu.VMEM((BPW,), jnp.int32),
            pltpu.VMEM((BPW,), jnp.int32),
            pltpu.VMEM((BPW,), jnp.int32),
            pltpu.VMEM((BPW, D), jnp.float32),
            pltpu.VMEM((BPW, D), jnp.float32),
            pltpu.VMEM((BPW, 16), jnp.float32),
            pltpu.VMEM((BPW, 16), jnp.float32),
            pltpu.VMEM((BPW,), jnp.float32),
            pltpu.VMEM((L * L,), jnp.float32),
            pltpu.SemaphoreType.DMA,
        ],
        compiler_params=pltpu.CompilerParams(
            needs_layout_passes=False,
            use_tc_tiling_on_sc=False,
        ),
    )(_mf_body)
    nv = user_bias.shape[0]
    pad = BROWS * 16 - nv
    ub2 = jnp.pad(user_bias.reshape(-1), (0, pad)).reshape(BROWS, 16)
    ib2 = jnp.pad(item_bias.reshape(-1), (0, pad)).reshape(BROWS, 16)
    return run(u, v, user_emb, item_emb, ub2, ib2)


def kernel(u, v, user_emb, item_emb, user_bias, item_bias):
    return _mf(u, v, user_emb, item_emb, user_bias, item_bias)


# speedup vs baseline: 1.0021x; 1.0021x over previous
import functools

import jax
import jax.numpy as jnp
from jax import lax
from jax.experimental import pallas as pl
from jax.experimental.pallas import tpu as pltpu
from jax.experimental.pallas import tpu_sc as plsc

B = 16384
D = 64
NC = 2
NS = 16
NW = NC * NS
BPW = B // NW
L = 16
BROWS = 62500


def _mf_body(u_hbm, v_hbm, ue_hbm, ie_hbm, ub_hbm, ib_hbm, out_hbm,
             uidx, vidx, ubg, vbg, urows, vrows, bub, bvb, obuf, pacc,
             sem):
    wid = lax.axis_index("s") * NC + lax.axis_index("c")
    base = wid * BPW

    pltpu.sync_copy(u_hbm.at[pl.ds(base, BPW)], uidx)
    pltpu.sync_copy(v_hbm.at[pl.ds(base, BPW)], vidx)

    cu = pltpu.async_copy(ue_hbm.at[uidx], urows, sem)
    cv = pltpu.async_copy(ie_hbm.at[vidx], vrows, sem)

    def shift(k, carry):
        ubg[pl.ds(k * L, L)] = lax.shift_right_logical(uidx[pl.ds(k * L, L)], 4)
        vbg[pl.ds(k * L, L)] = lax.shift_right_logical(vidx[pl.ds(k * L, L)], 4)
        return carry

    lax.fori_loop(0, BPW // L, shift, 0)

    cbu = pltpu.async_copy(ub_hbm.at[ubg], bub, sem)
    cbv = pltpu.async_copy(ib_hbm.at[vbg], bvb, sem)
    cu.wait()
    cv.wait()
    cbu.wait()
    cbv.wait()

    lanes = lax.iota(jnp.int32, L)
    lmask = jnp.full((L,), 15, jnp.int32)

    def block(kb, carry):
        for r in range(L):
            rr = kb * L + r
            acc = urows[rr, pl.ds(0, L)] * vrows[rr, pl.ds(0, L)]
            for c in range(1, D // L):
                acc = acc + urows[rr, pl.ds(c * L, L)] * vrows[rr, pl.ds(c * L, L)]
            pacc[pl.ds(r * L, L)] = acc
        tot = plsc.load_gather(pacc, [lanes * L])
        for t in range(1, L):
            tot = tot + plsc.load_gather(pacc, [lanes * L + t])
        rows = kb * L + lanes
        ulan = jnp.bitwise_and(uidx[pl.ds(kb * L, L)], lmask)
        vlan = jnp.bitwise_and(vidx[pl.ds(kb * L, L)], lmask)
        x = tot + plsc.load_gather(bub, [rows, ulan]) \
            + plsc.load_gather(bvb, [rows, vlan])
        obuf[pl.ds(kb * L, L)] = 1.0 / (1.0 + jnp.exp(-x))
        return carry

    lax.fori_loop(0, BPW // L, block, 0)

    pltpu.sync_copy(obuf, out_hbm.at[pl.ds(base, BPW)])


@jax.jit
def _mf(u, v, user_emb, item_emb, user_bias, item_bias):
    mesh = plsc.VectorSubcoreMesh(core_axis_name="c", subcore_axis_name="s")
    run = functools.partial(
        pl.kernel,
        mesh=mesh,
        out_type=jax.ShapeDtypeStruct((B,), jnp.float32),
        scratch_types=[
            pltpu.VMEM((BPW,), jnp.int32),
            pltpu.VMEM((BPW,), jnp.int32),
            pltpu.VMEM((BPW,), jnp.int32),
            pltpu.VMEM((BPW,), jnp.int32),
            pltpu.VMEM((BPW, D), jnp.float32),
            pltpu.VMEM((BPW, D), jnp.float32),
            pltpu.VMEM((BPW, 16), jnp.float32),
            pltpu.VMEM((BPW, 16), jnp.float32),
            pltpu.VMEM((BPW,), jnp.float32),
            pltpu.VMEM((L * L,), jnp.float32),
            pltpu.SemaphoreType.DMA,
        ],
        compiler_params=pltpu.CompilerParams(
            needs_layout_passes=False,
            use_tc_tiling_on_sc=False,
        ),
    )(_mf_body)
    ub2 = user_bias.reshape(BROWS, 16)
    ib2 = item_bias.reshape(BROWS, 16)
    return run(u, v, user_emb, item_emb, ub2, ib2)


def kernel(u, v, user_emb, item_emb, user_bias, item_bias):
    return _mf(u, v, user_emb, item_emb, user_bias, item_bias)
